# baseline (speedup 1.0000x reference)
import jax
import jax.numpy as jnp
from jax import lax
from jax.experimental import pallas as pl
from jax.experimental.pallas import tpu as pltpu

_EPS = 1e-20

_R = 128
_C = 100000
_RB = 8
_NRB = _R // _RB

_BIG_F32 = 1e9


def _argmax_body(x_ref, u_ref, idx_out):
    gcol = lax.broadcasted_iota(jnp.int32, (_RB, _C), 1).astype(jnp.float32)
    t = -jnp.log(u_ref[...] + _EPS) + _EPS
    f = jnp.exp(x_ref[...]) / t
    m = jnp.max(f, axis=1, keepdims=True)
    idx_out[...] = jnp.min(
        jnp.where(f == m, gcol, _BIG_F32), axis=1, keepdims=True
    )


def _onehot_body(idx_ref, out_ref):
    gcol = lax.broadcasted_iota(jnp.int32, (_RB, _C), 1).astype(jnp.float32)
    out_ref[...] = (gcol == idx_ref[...]).astype(jnp.float32)


@jax.jit
def kernel(x, U):
    idx = pl.pallas_call(
        _argmax_body,
        grid=(_NRB,),
        in_specs=[
            pl.BlockSpec((_RB, _C), lambda j: (j, 0)),
            pl.BlockSpec((_RB, _C), lambda j: (j, 0)),
        ],
        out_specs=pl.BlockSpec((_RB, 1), lambda j: (j, 0)),
        out_shape=jax.ShapeDtypeStruct((_R, 1), jnp.float32),
        compiler_params=pltpu.CompilerParams(
            dimension_semantics=("arbitrary",),
        ),
    )(x, U)

    return pl.pallas_call(
        _onehot_body,
        grid=(_NRB,),
        in_specs=[pl.BlockSpec((_RB, 1), lambda j: (j, 0))],
        out_specs=pl.BlockSpec((_RB, _C), lambda j: (j, 0)),
        out_shape=jax.ShapeDtypeStruct((_R, _C), jnp.float32),
        compiler_params=pltpu.CompilerParams(
            dimension_semantics=("arbitrary",),
        ),
    )(idx)
